# baseline (speedup 1.0000x reference)
import functools

import jax
import jax.numpy as jnp
from jax import lax
from jax.experimental import pallas as pl
from jax.experimental.pallas import tpu as pltpu
from jax.experimental.pallas import tpu_sc as plsc

VOCAB = 1000000
DIM = 64
B = 4096
L = 200

NC = 2
NS = 16
NW = NC * NS
LANES = 16

BBLK = B // NW
STEPS = L
NBUF = 4
GROUPS = STEPS // NBUF
NTB = 2
TPITCH = BBLK + 1


def _sc_gather(table2, idxT):
    mesh = plsc.VectorSubcoreMesh(core_axis_name="c", subcore_axis_name="s")

    @functools.partial(
        pl.kernel,
        mesh=mesh,
        out_type=jax.ShapeDtypeStruct((L, DIM // 8, B // 128, 8, 128), jnp.float32),
        compiler_params=pltpu.CompilerParams(needs_layout_passes=False),
        scratch_types=[pltpu.VMEM((BBLK,), jnp.int32) for _ in range(NBUF)]
        + [pltpu.VMEM((BBLK, 2 * DIM), jnp.float32) for _ in range(NBUF)]
        + [pltpu.VMEM((DIM // 8, 8, TPITCH), jnp.float32) for _ in range(NTB)]
        + [pltpu.SemaphoreType.DMA for _ in range(2 * NBUF + NTB)],
    )
    def k(table_hbm, idx_hbm, out_hbm, *scratch):
        ib = list(scratch[:NBUF])
        gb = list(scratch[NBUF : 2 * NBUF])
        tb = list(scratch[2 * NBUF : 2 * NBUF + NTB])
        isems = list(scratch[2 * NBUF + NTB : 3 * NBUF + NTB])
        gsems = list(scratch[3 * NBUF + NTB : 4 * NBUF + NTB])
        wsems = list(scratch[4 * NBUF + NTB : 4 * NBUF + 2 * NTB])

        wid = lax.axis_index("s") * NC + lax.axis_index("c")

        iota = lax.iota(jnp.int32, LANES)
        d0 = [(cg * LANES + iota) >> 3 for cg in range(DIM // LANES)]
        d1 = [(cg * LANES + iota) & 7 for cg in range(DIM // LANES)]
        def i_start(l, b):
            pltpu.async_copy(idx_hbm.at[l, pl.ds(wid * BBLK, BBLK)], ib[b], isems[b])

        def i_wait(l, b):
            pltpu.make_async_copy(
                idx_hbm.at[l, pl.ds(wid * BBLK, BBLK)], ib[b], isems[b]
            ).wait()

        def g_start(l, b):
            i_wait(l, b)
            pltpu.async_copy(table_hbm.at[ib[b]], gb[b], gsems[b])

        def g_wait(b):
            pltpu.make_async_copy(table_hbm.at[ib[b]], gb[b], gsems[b]).wait()

        def transpose(b, tbi):
            def perrow(b0, carry):
                vb0 = jnp.full((LANES,), b0, jnp.int32)
                vals = [
                    gb[b][b0, pl.ds(cg * LANES, LANES)]
                    for cg in range(DIM // LANES)
                ]
                for cg in range(DIM // LANES):
                    plsc.store_scatter(tb[tbi], [d0[cg], d1[cg], vb0], vals[cg])
                return carry

            lax.fori_loop(0, BBLK, perrow, 0)

        def w_start(l, tbi):
            pltpu.async_copy(
                tb[tbi].at[pl.ds(0, DIM // 8), pl.ds(0, 8), pl.ds(0, BBLK)],
                out_hbm.at[l, pl.ds(0, DIM // 8), wid],
                wsems[tbi],
            )

        def w_wait(l, tbi):
            pltpu.make_async_copy(
                tb[tbi].at[pl.ds(0, DIM // 8), pl.ds(0, 8), pl.ds(0, BBLK)],
                out_hbm.at[l, pl.ds(0, DIM // 8), wid],
                wsems[tbi],
            ).wait()

        for b in range(NBUF):
            i_start(b, b)
        for b in range(NBUF):
            g_start(b, b)

        def group(g, carry):
            l0 = g * NBUF
            for b in range(NBUF):
                tbi = b % NTB
                g_wait(b)
                lprev = l0 + b - NTB

                @pl.when(lprev >= 0)
                def _():
                    w_wait(lprev, tbi)

                transpose(b, tbi)
                w_start(l0 + b, tbi)
                nxt = l0 + NBUF + b

                @pl.when(nxt < STEPS)
                def _():
                    i_start(nxt, b)

            for b in range(NBUF):
                nxt = l0 + NBUF + b

                @pl.when(nxt < STEPS)
                def _():
                    g_start(nxt, b)

            return carry

        lax.fori_loop(0, GROUPS, group, 0)
        for t in range(NTB):
            w_wait(STEPS - NTB + t, (NBUF - NTB + t) % NTB)

    return k(table2, idxT)


def kernel(X_word, X_char, word_embed):
    del X_char
    idxT = X_word.T
    table2 = jnp.pad(word_embed, ((0, 0), (0, DIM)))
    out5d = _sc_gather(table2, idxT)
    return out5d.transpose(2, 4, 0, 1, 3).reshape(B, L, DIM)


# speedup vs baseline: 1.7093x; 1.7093x over previous
import functools

import jax
import jax.numpy as jnp
from jax import lax
from jax.experimental import pallas as pl
from jax.experimental.pallas import tpu as pltpu
from jax.experimental.pallas import tpu_sc as plsc

VOCAB = 1000000
DIM = 64
B = 4096
L = 200

NC = 2
NS = 16
NW = NC * NS
LANES = 16

BBLK = B // NW
STEPS = L
NBUF = 4
GROUPS = STEPS // NBUF
NTB = 2
TPITCH = BBLK + 1


def _sc_gather(table2, idxT):
    mesh = plsc.VectorSubcoreMesh(core_axis_name="c", subcore_axis_name="s")

    @functools.partial(
        pl.kernel,
        mesh=mesh,
        out_type=jax.ShapeDtypeStruct((L, DIM // 8, B // 128, 8, 128), jnp.float32),
        compiler_params=pltpu.CompilerParams(
            use_tc_tiling_on_sc=False, needs_layout_passes=False
        ),
        scratch_types=[pltpu.VMEM((BBLK,), jnp.int32) for _ in range(NBUF)]
        + [pltpu.VMEM((BBLK, DIM), jnp.float32) for _ in range(NBUF)]
        + [pltpu.VMEM((DIM // 8, 8, TPITCH), jnp.float32) for _ in range(NTB)]
        + [pltpu.SemaphoreType.DMA for _ in range(2 * NBUF + NTB)],
    )
    def k(table_hbm, idx_hbm, out_hbm, *scratch):
        ib = list(scratch[:NBUF])
        gb = list(scratch[NBUF : 2 * NBUF])
        tb = list(scratch[2 * NBUF : 2 * NBUF + NTB])
        isems = list(scratch[2 * NBUF + NTB : 3 * NBUF + NTB])
        gsems = list(scratch[3 * NBUF + NTB : 4 * NBUF + NTB])
        wsems = list(scratch[4 * NBUF + NTB : 4 * NBUF + 2 * NTB])

        wid = lax.axis_index("s") * NC + lax.axis_index("c")

        iota = lax.iota(jnp.int32, LANES)
        d0 = [(cg * LANES + iota) >> 3 for cg in range(DIM // LANES)]
        d1 = [(cg * LANES + iota) & 7 for cg in range(DIM // LANES)]
        def i_start(l, b):
            pltpu.async_copy(idx_hbm.at[l, pl.ds(wid * BBLK, BBLK)], ib[b], isems[b])

        def i_wait(l, b):
            pltpu.make_async_copy(
                idx_hbm.at[l, pl.ds(wid * BBLK, BBLK)], ib[b], isems[b]
            ).wait()

        def g_start(l, b):
            i_wait(l, b)
            pltpu.async_copy(table_hbm.at[ib[b]], gb[b], gsems[b])

        def g_wait(b):
            pltpu.make_async_copy(table_hbm.at[ib[b]], gb[b], gsems[b]).wait()

        def transpose(b, tbi):
            def perrow(b0, carry):
                vb0 = jnp.full((LANES,), b0, jnp.int32)
                vals = [
                    gb[b][b0, pl.ds(cg * LANES, LANES)]
                    for cg in range(DIM // LANES)
                ]
                for cg in range(DIM // LANES):
                    plsc.store_scatter(tb[tbi], [d0[cg], d1[cg], vb0], vals[cg])
                return carry

            lax.fori_loop(0, BBLK, perrow, 0)

        def w_start(l, tbi):
            pltpu.async_copy(
                tb[tbi].at[pl.ds(0, DIM // 8), pl.ds(0, 8), pl.ds(0, BBLK)],
                out_hbm.at[l, pl.ds(0, DIM // 8), wid],
                wsems[tbi],
            )

        def w_wait(l, tbi):
            pltpu.make_async_copy(
                tb[tbi].at[pl.ds(0, DIM // 8), pl.ds(0, 8), pl.ds(0, BBLK)],
                out_hbm.at[l, pl.ds(0, DIM // 8), wid],
                wsems[tbi],
            ).wait()

        for b in range(NBUF):
            i_start(b, b)
        for b in range(NBUF):
            g_start(b, b)

        def group(g, carry):
            l0 = g * NBUF
            for b in range(NBUF):
                tbi = b % NTB
                g_wait(b)
                lprev = l0 + b - NTB

                @pl.when(lprev >= 0)
                def _():
                    w_wait(lprev, tbi)

                transpose(b, tbi)
                w_start(l0 + b, tbi)
                nxt = l0 + NBUF + b

                @pl.when(nxt < STEPS)
                def _():
                    i_start(nxt, b)

            for b in range(NBUF):
                nxt = l0 + NBUF + b

                @pl.when(nxt < STEPS)
                def _():
                    g_start(nxt, b)

            return carry

        lax.fori_loop(0, GROUPS, group, 0)
        for t in range(NTB):
            w_wait(STEPS - NTB + t, (NBUF - NTB + t) % NTB)

    return k(table2, idxT)


def kernel(X_word, X_char, word_embed):
    del X_char
    idxT = X_word.T
    out5d = _sc_gather(word_embed, idxT)
    return out5d.transpose(2, 4, 0, 1, 3).reshape(B, L, DIM)


# speedup vs baseline: 1.7104x; 1.0006x over previous
import functools

import jax
import jax.numpy as jnp
from jax import lax
from jax.experimental import pallas as pl
from jax.experimental.pallas import tpu as pltpu
from jax.experimental.pallas import tpu_sc as plsc

VOCAB = 1000000
DIM = 64
B = 4096
L = 200

NC = 2
NS = 16
NW = NC * NS
LANES = 16

BBLK = B // NW
STEPS = L
NBUF = 4
GROUPS = STEPS // NBUF
NTB = 2
TPITCH = BBLK + 1


def _sc_gather(table2, idxT):
    mesh = plsc.VectorSubcoreMesh(core_axis_name="c", subcore_axis_name="s")

    @functools.partial(
        pl.kernel,
        mesh=mesh,
        out_type=jax.ShapeDtypeStruct((L, DIM // 8, B // 128, 8, 128), jnp.float32),
        compiler_params=pltpu.CompilerParams(
            use_tc_tiling_on_sc=False, needs_layout_passes=False
        ),
        scratch_types=[pltpu.VMEM((BBLK,), jnp.int32) for _ in range(NBUF)]
        + [pltpu.VMEM((BBLK, DIM), jnp.float32) for _ in range(NBUF)]
        + [pltpu.VMEM((DIM // 8, 8, TPITCH), jnp.float32) for _ in range(NTB)]
        + [pltpu.SemaphoreType.DMA for _ in range(2 * NBUF + NTB)],
    )
    def k(table_hbm, idx_hbm, out_hbm, *scratch):
        ib = list(scratch[:NBUF])
        gb = list(scratch[NBUF : 2 * NBUF])
        tb = list(scratch[2 * NBUF : 2 * NBUF + NTB])
        isems = list(scratch[2 * NBUF + NTB : 3 * NBUF + NTB])
        gsems = list(scratch[3 * NBUF + NTB : 4 * NBUF + NTB])
        wsems = list(scratch[4 * NBUF + NTB : 4 * NBUF + 2 * NTB])

        wid = lax.axis_index("s") * NC + lax.axis_index("c")

        iota = lax.iota(jnp.int32, LANES)
        d0 = [(cg * LANES + iota) >> 3 for cg in range(DIM // LANES)]
        d1 = [(cg * LANES + iota) & 7 for cg in range(DIM // LANES)]
        def i_start(l, b):
            pltpu.async_copy(idx_hbm.at[l, pl.ds(wid * BBLK, BBLK)], ib[b], isems[b])

        def i_wait(l, b):
            pltpu.make_async_copy(
                idx_hbm.at[l, pl.ds(wid * BBLK, BBLK)], ib[b], isems[b]
            ).wait()

        def g_start(l, b):
            i_wait(l, b)
            pltpu.async_copy(table_hbm.at[ib[b]], gb[b], gsems[b])

        def g_wait(b):
            pltpu.make_async_copy(table_hbm.at[ib[b]], gb[b], gsems[b]).wait()

        ROWU = 4

        def transpose(b, tbi):
            def perrows(r, carry):
                b0 = r * ROWU
                for u in range(ROWU):
                    vb0 = jnp.full((LANES,), b0 + u, jnp.int32)
                    vals = [
                        gb[b][b0 + u, pl.ds(cg * LANES, LANES)]
                        for cg in range(DIM // LANES)
                    ]
                    for cg in range(DIM // LANES):
                        plsc.store_scatter(tb[tbi], [d0[cg], d1[cg], vb0], vals[cg])
                return carry

            lax.fori_loop(0, BBLK // ROWU, perrows, 0)

        def w_start(l, tbi):
            pltpu.async_copy(
                tb[tbi].at[pl.ds(0, DIM // 8), pl.ds(0, 8), pl.ds(0, BBLK)],
                out_hbm.at[l, pl.ds(0, DIM // 8), wid],
                wsems[tbi],
            )

        def w_wait(l, tbi):
            pltpu.make_async_copy(
                tb[tbi].at[pl.ds(0, DIM // 8), pl.ds(0, 8), pl.ds(0, BBLK)],
                out_hbm.at[l, pl.ds(0, DIM // 8), wid],
                wsems[tbi],
            ).wait()

        for b in range(NBUF):
            i_start(b, b)
        for b in range(NBUF):
            g_start(b, b)

        def group(g, carry):
            l0 = g * NBUF
            for b in range(NBUF):
                tbi = b % NTB
                g_wait(b)
                lprev = l0 + b - NTB

                @pl.when(lprev >= 0)
                def _():
                    w_wait(lprev, tbi)

                transpose(b, tbi)
                w_start(l0 + b, tbi)
                nxt = l0 + NBUF + b

                @pl.when(nxt < STEPS)
                def _():
                    i_start(nxt, b)

            for b in range(NBUF):
                nxt = l0 + NBUF + b

                @pl.when(nxt < STEPS)
                def _():
                    g_start(nxt, b)

            return carry

        lax.fori_loop(0, GROUPS, group, 0)
        for t in range(NTB):
            w_wait(STEPS - NTB + t, (NBUF - NTB + t) % NTB)

    return k(table2, idxT)


def kernel(X_word, X_char, word_embed):
    del X_char
    idxT = X_word.T
    out5d = _sc_gather(word_embed, idxT)
    return out5d.transpose(2, 4, 0, 1, 3).reshape(B, L, DIM)
